# baseline (speedup 1.0000x reference)
import functools

import jax
import jax.numpy as jnp
from jax import lax
from jax.experimental import pallas as pl
from jax.experimental.pallas import tpu as pltpu
from jax.experimental.pallas import tpu_sc as plsc

N = 10000
D = 128
E = 320000
DH = 64

N_P = 10240
ROWS_PER_TILE = N_P // 16

EA_PER_TILE = 10112
EA_TOT = 32 * EA_PER_TILE

CHUNK = 128
NCHUNK = 160
EB_PER_TILE = NCHUNK * CHUNK
EB_TOT = 16 * EB_PER_TILE

def _deg_body(dst_hbm, cnt_hbm, idx_v, cnt_v):
    c = lax.axis_index("c")
    s = lax.axis_index("s")
    wid = c * 16 + s
    pltpu.sync_copy(dst_hbm.at[wid], idx_v)

    zero16 = jnp.zeros((16,), jnp.float32)

    def zero_step(i, carry):
        cnt_v[pl.ds(i * 16, 16)] = zero16
        return carry

    lax.fori_loop(0, N_P // 16, zero_step, 0, unroll=8)

    ones16 = jnp.ones((16,), jnp.float32)

    def acc_step(i, carry):
        idx = idx_v[pl.ds(i * 16, 16)]
        plsc.addupdate_scatter(cnt_v, [idx], ones16)
        return carry

    lax.fori_loop(0, EA_PER_TILE // 16, acc_step, 0, unroll=8)
    pltpu.sync_copy(cnt_v, cnt_hbm.at[wid])


@functools.lru_cache(maxsize=None)
def _deg_call():
    mesh = plsc.VectorSubcoreMesh(
        core_axis_name="c", subcore_axis_name="s", num_cores=2, num_subcores=16
    )
    return pl.kernel(
        _deg_body,
        out_type=jax.ShapeDtypeStruct((32, N_P), jnp.float32),
        mesh=mesh,
        compiler_params=pltpu.CompilerParams(needs_layout_passes=False),
        scratch_types=[
            pltpu.VMEM((EA_PER_TILE,), jnp.int32),
            pltpu.VMEM((N_P,), jnp.float32),
        ],
    )


def _segsum_body(y_hbm, src_hbm, dst_hbm, out_hbm,
                 out_sh, idx_s, idx_d, buf_a, buf_b, sem_a, sem_b):
    c = lax.axis_index("c")
    s = lax.axis_index("s")
    rows = pl.ds(s * ROWS_PER_TILE, ROWS_PER_TILE)
    myrows = pl.ds(c * N_P + s * ROWS_PER_TILE, ROWS_PER_TILE)

    pltpu.sync_copy(y_hbm.at[myrows], out_sh.at[rows])
    pltpu.sync_copy(src_hbm.at[c, s], idx_s)
    pltpu.sync_copy(dst_hbm.at[s], idx_d)
    plsc.subcore_barrier()

    pltpu.async_copy(y_hbm.at[idx_s.at[0]], buf_a, sem_a)
    pltpu.async_copy(y_hbm.at[idx_s.at[1]], buf_b, sem_b)

    def step(jj, carry):
        j = 2 * jj
        pltpu.make_async_copy(y_hbm.at[idx_s.at[j]], buf_a, sem_a).wait()
        pltpu.sync_copy(buf_a, out_sh.at[idx_d.at[j]], add=True)

        @pl.when(j + 2 < NCHUNK)
        def _():
            pltpu.async_copy(y_hbm.at[idx_s.at[j + 2]], buf_a, sem_a)

        pltpu.make_async_copy(y_hbm.at[idx_s.at[j + 1]], buf_b, sem_b).wait()
        pltpu.sync_copy(buf_b, out_sh.at[idx_d.at[j + 1]], add=True)

        @pl.when(j + 3 < NCHUNK)
        def _():
            pltpu.async_copy(y_hbm.at[idx_s.at[j + 3]], buf_b, sem_b)

        return carry

    lax.fori_loop(0, NCHUNK // 2, step, 0)
    plsc.subcore_barrier()

    pltpu.sync_copy(out_sh.at[rows], out_hbm.at[c, rows])


@functools.lru_cache(maxsize=None)
def _segsum_call():
    mesh = plsc.VectorSubcoreMesh(
        core_axis_name="c", subcore_axis_name="s", num_cores=2, num_subcores=16
    )
    return pl.kernel(
        _segsum_body,
        out_type=jax.ShapeDtypeStruct((2, N_P, DH), jnp.float32),
        mesh=mesh,
        compiler_params=pltpu.CompilerParams(
            needs_layout_passes=False, use_tc_tiling_on_sc=False
        ),
        scratch_types=[
            pltpu.VMEM_SHARED((N_P, DH), jnp.float32),
            pltpu.VMEM((NCHUNK, CHUNK), jnp.int32),
            pltpu.VMEM((NCHUNK, CHUNK), jnp.int32),
            pltpu.VMEM((CHUNK, DH), jnp.float32),
            pltpu.VMEM((CHUNK, DH), jnp.float32),
            pltpu.SemaphoreType.DMA,
            pltpu.SemaphoreType.DMA,
        ],
    )


_BLK = 128


def _tc1_body(x_ref, w_ref, b_ref, cnt_ref, y_ref, dinv_ref):
    i = pl.program_id(0)
    xw = jnp.dot(x_ref[...], w_ref[...], preferred_element_type=jnp.float32)
    xw = xw + b_ref[...]
    ones = jnp.ones((32, 1), jnp.float32)
    deg = lax.dot_general(cnt_ref[...], ones, (((0,), (0,)), ((), ())),
                          preferred_element_type=jnp.float32)
    rowid = i * _BLK + lax.broadcasted_iota(jnp.int32, (_BLK, 1), 0)
    deg = deg + jnp.where(rowid < N, 1.0, 0.0)
    dinv = jnp.where(deg > 0, lax.rsqrt(deg), 0.0)
    y = xw * dinv
    y_ref[0] = y[:, :DH]
    y_ref[1] = y[:, DH:]
    dinv_ref[...] = dinv


def _tc2_body(s1_ref, dinv_ref, w_ref, b_ref, y_ref):
    s1 = jnp.concatenate([s1_ref[0], s1_ref[1]], axis=-1)
    dinv = dinv_ref[...]
    h = jnp.maximum(s1 * dinv, 0.0)
    xw = jnp.dot(h, w_ref[...], preferred_element_type=jnp.float32)
    xw = xw + b_ref[...]
    y = xw * dinv
    y_ref[0] = y[:, :DH]
    y_ref[1] = y[:, DH:]


def _tc3_body(s2_ref, dinv_ref, k_ref, wk_ref, bk_ref, alpha_ref, o_ref):
    s2 = jnp.concatenate([s2_ref[0], s2_ref[1]], axis=-1)
    x2 = s2 * dinv_ref[...]
    kp = jnp.dot(k_ref[...], wk_ref[...], preferred_element_type=jnp.float32)
    kp = kp + bk_ref[...]
    a = alpha_ref[0, 0]
    o_ref[...] = a * x2 + (1.0 - a) * kp


def kernel(graph_feature, k_node_pred, edge_index, W1, b1, W2, b2, Wk, bk, alpha):
    i32 = jnp.int32
    src = edge_index[0].astype(i32)
    dst = edge_index[1].astype(i32)

    pad_a = jnp.full((EA_TOT - E,), N, i32)
    dst_a = jnp.concatenate([dst, pad_a]).reshape(32, EA_PER_TILE)
    pad_b = jnp.full((EB_TOT - E,), N, i32)
    src_b = jnp.concatenate([src, pad_b]).reshape(16, NCHUNK, CHUNK)
    src_bc = jnp.stack([src_b, src_b + N_P])
    dst_b = jnp.concatenate([dst, pad_b]).reshape(16, NCHUNK, CHUNK)

    x_pad = jnp.pad(graph_feature, ((0, N_P - N), (0, 0)))
    b1r = b1.reshape(1, D)
    b2r = b2.reshape(1, D)
    bkr = bk.reshape(1, D)
    alpha_arr = jnp.reshape(alpha, (1, 1)).astype(jnp.float32)

    cnt = _deg_call()(dst_a)

    grid = N_P // _BLK
    y1, dinv = pl.pallas_call(
        _tc1_body,
        grid=(grid,),
        in_specs=[
            pl.BlockSpec((_BLK, D), lambda i: (i, 0)),
            pl.BlockSpec((D, D), lambda i: (0, 0)),
            pl.BlockSpec((1, D), lambda i: (0, 0)),
            pl.BlockSpec((32, _BLK), lambda i: (0, i)),
        ],
        out_specs=[
            pl.BlockSpec((2, _BLK, DH), lambda i: (0, i, 0)),
            pl.BlockSpec((_BLK, 1), lambda i: (i, 0)),
        ],
        out_shape=[
            jax.ShapeDtypeStruct((2, N_P, DH), jnp.float32),
            jax.ShapeDtypeStruct((N_P, 1), jnp.float32),
        ],
    )(x_pad, W1, b1r, cnt)

    s1 = _segsum_call()(y1.reshape(2 * N_P, DH), src_bc, dst_b)

    y2 = pl.pallas_call(
        _tc2_body,
        grid=(grid,),
        in_specs=[
            pl.BlockSpec((2, _BLK, DH), lambda i: (0, i, 0)),
            pl.BlockSpec((_BLK, 1), lambda i: (i, 0)),
            pl.BlockSpec((D, D), lambda i: (0, 0)),
            pl.BlockSpec((1, D), lambda i: (0, 0)),
        ],
        out_specs=pl.BlockSpec((2, _BLK, DH), lambda i: (0, i, 0)),
        out_shape=jax.ShapeDtypeStruct((2, N_P, DH), jnp.float32),
    )(s1, dinv, W2, b2r)

    s2 = _segsum_call()(y2.reshape(2 * N_P, DH), src_bc, dst_b)

    blk3 = 200
    out = pl.pallas_call(
        _tc3_body,
        grid=(N // blk3,),
        in_specs=[
            pl.BlockSpec((2, blk3, DH), lambda i: (0, i, 0)),
            pl.BlockSpec((blk3, 1), lambda i: (i, 0)),
            pl.BlockSpec((blk3, 64), lambda i: (i, 0)),
            pl.BlockSpec((64, D), lambda i: (0, 0)),
            pl.BlockSpec((1, D), lambda i: (0, 0)),
            pl.BlockSpec((1, 1), lambda i: (0, 0)),
        ],
        out_specs=pl.BlockSpec((blk3, D), lambda i: (i, 0)),
        out_shape=jax.ShapeDtypeStruct((N, D), jnp.float32),
    )(s2, dinv, k_node_pred, Wk, bkr, alpha_arr)

    return out


# speedup vs baseline: 1.4668x; 1.4668x over previous
import functools

import jax
import jax.numpy as jnp
from jax import lax
from jax.experimental import pallas as pl
from jax.experimental.pallas import tpu as pltpu
from jax.experimental.pallas import tpu_sc as plsc

N = 10000
D = 128
E = 320000
DH = 64

N_P = 10240
ROWS_PER_TILE = N_P // 16

EA_PER_TILE = 10112
EA_TOT = 32 * EA_PER_TILE

CHUNK = 128
NCHUNK = 160
HALF = 80
EB_PER_TILE = NCHUNK * CHUNK
EB_TOT = 16 * EB_PER_TILE

def _deg_body(dst_hbm, cnt_hbm, idx_v, cnt_v):
    c = lax.axis_index("c")
    s = lax.axis_index("s")
    wid = c * 16 + s
    pltpu.sync_copy(dst_hbm.at[wid], idx_v)

    zero16 = jnp.zeros((16,), jnp.float32)

    def zero_step(i, carry):
        cnt_v[pl.ds(i * 16, 16)] = zero16
        return carry

    lax.fori_loop(0, N_P // 16, zero_step, 0, unroll=8)

    ones16 = jnp.ones((16,), jnp.float32)

    def acc_step(i, carry):
        idx = idx_v[pl.ds(i * 16, 16)]
        plsc.addupdate_scatter(cnt_v, [idx], ones16)
        return carry

    lax.fori_loop(0, EA_PER_TILE // 16, acc_step, 0, unroll=8)
    pltpu.sync_copy(cnt_v, cnt_hbm.at[wid])


@functools.lru_cache(maxsize=None)
def _deg_call():
    mesh = plsc.VectorSubcoreMesh(
        core_axis_name="c", subcore_axis_name="s", num_cores=2, num_subcores=16
    )
    return pl.kernel(
        _deg_body,
        out_type=jax.ShapeDtypeStruct((32, N_P), jnp.float32),
        mesh=mesh,
        compiler_params=pltpu.CompilerParams(needs_layout_passes=False),
        scratch_types=[
            pltpu.VMEM((EA_PER_TILE,), jnp.int32),
            pltpu.VMEM((N_P,), jnp.float32),
        ],
    )


def _segsum_body(y_hbm, src_hbm, dst_hbm, out_hbm,
                 y_sh, out_sh, idx_s, idx_d, buf_a, buf_b, sem_a, sem_b):
    c = lax.axis_index("c")
    s = lax.axis_index("s")
    rows = pl.ds(s * ROWS_PER_TILE, ROWS_PER_TILE)

    pltpu.sync_copy(y_hbm.at[c, rows], y_sh.at[rows])
    pltpu.sync_copy(y_hbm.at[c, rows], out_sh.at[rows])
    plsc.subcore_barrier()

    def half(h, carry):
        pltpu.sync_copy(src_hbm.at[s, pl.ds(h * HALF, HALF)], idx_s)
        pltpu.sync_copy(dst_hbm.at[s, pl.ds(h * HALF, HALF)], idx_d)

        pltpu.async_copy(y_sh.at[idx_s.at[0]], buf_a, sem_a)
        pltpu.async_copy(y_sh.at[idx_s.at[1]], buf_b, sem_b)

        def step(jj, carry2):
            j = 2 * jj
            pltpu.make_async_copy(y_sh.at[idx_s.at[j]], buf_a, sem_a).wait()
            pltpu.sync_copy(buf_a, out_sh.at[idx_d.at[j]], add=True)

            @pl.when(j + 2 < HALF)
            def _():
                pltpu.async_copy(y_sh.at[idx_s.at[j + 2]], buf_a, sem_a)

            pltpu.make_async_copy(y_sh.at[idx_s.at[j + 1]], buf_b, sem_b).wait()
            pltpu.sync_copy(buf_b, out_sh.at[idx_d.at[j + 1]], add=True)

            @pl.when(j + 3 < HALF)
            def _():
                pltpu.async_copy(y_sh.at[idx_s.at[j + 3]], buf_b, sem_b)

            return carry2

        lax.fori_loop(0, HALF // 2, step, 0)
        return carry

    lax.fori_loop(0, 2, half, 0)
    plsc.subcore_barrier()

    pltpu.sync_copy(out_sh.at[rows], out_hbm.at[c, rows])


@functools.lru_cache(maxsize=None)
def _segsum_call():
    mesh = plsc.VectorSubcoreMesh(
        core_axis_name="c", subcore_axis_name="s", num_cores=2, num_subcores=16
    )
    return pl.kernel(
        _segsum_body,
        out_type=jax.ShapeDtypeStruct((2, N_P, DH), jnp.float32),
        mesh=mesh,
        compiler_params=pltpu.CompilerParams(
            needs_layout_passes=False, use_tc_tiling_on_sc=False
        ),
        scratch_types=[
            pltpu.VMEM_SHARED((N_P, DH), jnp.float32),
            pltpu.VMEM_SHARED((N_P, DH), jnp.float32),
            pltpu.VMEM((HALF, CHUNK), jnp.int32),
            pltpu.VMEM((HALF, CHUNK), jnp.int32),
            pltpu.VMEM((CHUNK, DH), jnp.float32),
            pltpu.VMEM((CHUNK, DH), jnp.float32),
            pltpu.SemaphoreType.DMA,
            pltpu.SemaphoreType.DMA,
        ],
    )


_BLK = 128


def _tc1_body(x_ref, w_ref, b_ref, cnt_ref, y_ref, dinv_ref):
    i = pl.program_id(0)
    xw = jnp.dot(x_ref[...], w_ref[...], preferred_element_type=jnp.float32)
    xw = xw + b_ref[...]
    ones = jnp.ones((32, 1), jnp.float32)
    deg = lax.dot_general(cnt_ref[...], ones, (((0,), (0,)), ((), ())),
                          preferred_element_type=jnp.float32)
    rowid = i * _BLK + lax.broadcasted_iota(jnp.int32, (_BLK, 1), 0)
    deg = deg + jnp.where(rowid < N, 1.0, 0.0)
    dinv = jnp.where(deg > 0, lax.rsqrt(deg), 0.0)
    y = xw * dinv
    y_ref[0] = y[:, :DH]
    y_ref[1] = y[:, DH:]
    dinv_ref[...] = dinv


def _tc2_body(s1_ref, dinv_ref, w_ref, b_ref, y_ref):
    s1 = jnp.concatenate([s1_ref[0], s1_ref[1]], axis=-1)
    dinv = dinv_ref[...]
    h = jnp.maximum(s1 * dinv, 0.0)
    xw = jnp.dot(h, w_ref[...], preferred_element_type=jnp.float32)
    xw = xw + b_ref[...]
    y = xw * dinv
    y_ref[0] = y[:, :DH]
    y_ref[1] = y[:, DH:]


def _tc3_body(s2_ref, dinv_ref, k_ref, wk_ref, bk_ref, alpha_ref, o_ref):
    s2 = jnp.concatenate([s2_ref[0], s2_ref[1]], axis=-1)
    x2 = s2 * dinv_ref[...]
    kp = jnp.dot(k_ref[...], wk_ref[...], preferred_element_type=jnp.float32)
    kp = kp + bk_ref[...]
    a = alpha_ref[0, 0]
    o_ref[...] = a * x2 + (1.0 - a) * kp


def kernel(graph_feature, k_node_pred, edge_index, W1, b1, W2, b2, Wk, bk, alpha):
    i32 = jnp.int32
    src = edge_index[0].astype(i32)
    dst = edge_index[1].astype(i32)

    pad_a = jnp.full((EA_TOT - E,), N, i32)
    dst_a = jnp.concatenate([dst, pad_a]).reshape(32, EA_PER_TILE)
    pad_b = jnp.full((EB_TOT - E,), N, i32)
    src_b = jnp.concatenate([src, pad_b]).reshape(16, NCHUNK, CHUNK)
    dst_b = jnp.concatenate([dst, pad_b]).reshape(16, NCHUNK, CHUNK)

    x_pad = jnp.pad(graph_feature, ((0, N_P - N), (0, 0)))
    b1r = b1.reshape(1, D)
    b2r = b2.reshape(1, D)
    bkr = bk.reshape(1, D)
    alpha_arr = jnp.reshape(alpha, (1, 1)).astype(jnp.float32)

    cnt = _deg_call()(dst_a)

    grid = N_P // _BLK
    y1, dinv = pl.pallas_call(
        _tc1_body,
        grid=(grid,),
        in_specs=[
            pl.BlockSpec((_BLK, D), lambda i: (i, 0)),
            pl.BlockSpec((D, D), lambda i: (0, 0)),
            pl.BlockSpec((1, D), lambda i: (0, 0)),
            pl.BlockSpec((32, _BLK), lambda i: (0, i)),
        ],
        out_specs=[
            pl.BlockSpec((2, _BLK, DH), lambda i: (0, i, 0)),
            pl.BlockSpec((_BLK, 1), lambda i: (i, 0)),
        ],
        out_shape=[
            jax.ShapeDtypeStruct((2, N_P, DH), jnp.float32),
            jax.ShapeDtypeStruct((N_P, 1), jnp.float32),
        ],
    )(x_pad, W1, b1r, cnt)

    s1 = _segsum_call()(y1, src_b, dst_b)

    y2 = pl.pallas_call(
        _tc2_body,
        grid=(grid,),
        in_specs=[
            pl.BlockSpec((2, _BLK, DH), lambda i: (0, i, 0)),
            pl.BlockSpec((_BLK, 1), lambda i: (i, 0)),
            pl.BlockSpec((D, D), lambda i: (0, 0)),
            pl.BlockSpec((1, D), lambda i: (0, 0)),
        ],
        out_specs=pl.BlockSpec((2, _BLK, DH), lambda i: (0, i, 0)),
        out_shape=jax.ShapeDtypeStruct((2, N_P, DH), jnp.float32),
    )(s1, dinv, W2, b2r)

    s2 = _segsum_call()(y2, src_b, dst_b)

    blk3 = 200
    out = pl.pallas_call(
        _tc3_body,
        grid=(N // blk3,),
        in_specs=[
            pl.BlockSpec((2, blk3, DH), lambda i: (0, i, 0)),
            pl.BlockSpec((blk3, 1), lambda i: (i, 0)),
            pl.BlockSpec((blk3, 64), lambda i: (i, 0)),
            pl.BlockSpec((64, D), lambda i: (0, 0)),
            pl.BlockSpec((1, D), lambda i: (0, 0)),
            pl.BlockSpec((1, 1), lambda i: (0, 0)),
        ],
        out_specs=pl.BlockSpec((blk3, D), lambda i: (i, 0)),
        out_shape=jax.ShapeDtypeStruct((N, D), jnp.float32),
    )(s2, dinv, k_node_pred, Wk, bkr, alpha_arr)

    return out


# speedup vs baseline: 2.0625x; 1.4062x over previous
import functools

import jax
import jax.numpy as jnp
from jax import lax
from jax.experimental import pallas as pl
from jax.experimental.pallas import tpu as pltpu
from jax.experimental.pallas import tpu_sc as plsc

N = 10000
D = 128
E = 320000
DH = 64

N_P = 10240
ROWS_PER_TILE = N_P // 16

EA_PER_TILE = 10000
EA_TOT = 32 * EA_PER_TILE

CHUNK = 125
NCHUNK = 160
HALF = 80
EB_PER_TILE = NCHUNK * CHUNK
EB_TOT = 16 * EB_PER_TILE

def _deg_body(dst_hbm, cnt_hbm, idx_v, cnt_v):
    c = lax.axis_index("c")
    s = lax.axis_index("s")
    wid = c * 16 + s
    pltpu.sync_copy(dst_hbm.at[wid], idx_v)

    zero16 = jnp.zeros((16,), jnp.float32)

    def zero_step(i, carry):
        cnt_v[pl.ds(i * 16, 16)] = zero16
        return carry

    lax.fori_loop(0, N_P // 16, zero_step, 0, unroll=8)

    ones16 = jnp.ones((16,), jnp.float32)

    def acc_step(i, carry):
        idx = idx_v[pl.ds(i * 16, 16)]
        plsc.addupdate_scatter(cnt_v, [idx], ones16)
        return carry

    lax.fori_loop(0, EA_PER_TILE // 16, acc_step, 0, unroll=8)
    pltpu.sync_copy(cnt_v, cnt_hbm.at[wid])


@functools.lru_cache(maxsize=None)
def _deg_call():
    mesh = plsc.VectorSubcoreMesh(
        core_axis_name="c", subcore_axis_name="s", num_cores=2, num_subcores=16
    )
    return pl.kernel(
        _deg_body,
        out_type=jax.ShapeDtypeStruct((32, N_P), jnp.float32),
        mesh=mesh,
        compiler_params=pltpu.CompilerParams(needs_layout_passes=False),
        scratch_types=[
            pltpu.VMEM((EA_PER_TILE,), jnp.int32),
            pltpu.VMEM((N_P,), jnp.float32),
        ],
    )


def _segsum_body(y_hbm, src_hbm, dst_hbm, out_hbm,
                 y_sh, out_sh, idx_s, idx_d, buf_a, buf_b, sem_a, sem_b):
    c = lax.axis_index("c")
    s = lax.axis_index("s")
    rows = pl.ds(s * ROWS_PER_TILE, ROWS_PER_TILE)

    cols = pl.ds(c * DH, DH)
    pltpu.sync_copy(y_hbm.at[rows, cols], y_sh.at[rows])
    pltpu.sync_copy(y_hbm.at[rows, cols], out_sh.at[rows])
    plsc.subcore_barrier()

    def half(h, carry):
        pltpu.sync_copy(src_hbm.at[s, pl.ds(h * HALF, HALF)], idx_s)
        pltpu.sync_copy(dst_hbm.at[s, pl.ds(h * HALF, HALF)], idx_d)

        pltpu.async_copy(y_sh.at[idx_s.at[0]], buf_a, sem_a)
        pltpu.async_copy(y_sh.at[idx_s.at[1]], buf_b, sem_b)

        def step(jj, carry2):
            j = 2 * jj
            pltpu.make_async_copy(y_sh.at[idx_s.at[j]], buf_a, sem_a).wait()
            pltpu.sync_copy(buf_a, out_sh.at[idx_d.at[j]], add=True)

            @pl.when(j + 2 < HALF)
            def _():
                pltpu.async_copy(y_sh.at[idx_s.at[j + 2]], buf_a, sem_a)

            pltpu.make_async_copy(y_sh.at[idx_s.at[j + 1]], buf_b, sem_b).wait()
            pltpu.sync_copy(buf_b, out_sh.at[idx_d.at[j + 1]], add=True)

            @pl.when(j + 3 < HALF)
            def _():
                pltpu.async_copy(y_sh.at[idx_s.at[j + 3]], buf_b, sem_b)

            return carry2

        lax.fori_loop(0, HALF // 2, step, 0)
        return carry

    lax.fori_loop(0, 2, half, 0)
    plsc.subcore_barrier()

    pltpu.sync_copy(out_sh.at[rows], out_hbm.at[rows, cols])


@functools.lru_cache(maxsize=None)
def _segsum_call():
    mesh = plsc.VectorSubcoreMesh(
        core_axis_name="c", subcore_axis_name="s", num_cores=2, num_subcores=16
    )
    return pl.kernel(
        _segsum_body,
        out_type=jax.ShapeDtypeStruct((N_P, D), jnp.float32),
        mesh=mesh,
        compiler_params=pltpu.CompilerParams(
            needs_layout_passes=False, use_tc_tiling_on_sc=False
        ),
        scratch_types=[
            pltpu.VMEM_SHARED((N_P, DH), jnp.float32),
            pltpu.VMEM_SHARED((N_P, DH), jnp.float32),
            pltpu.VMEM((HALF, CHUNK), jnp.int32),
            pltpu.VMEM((HALF, CHUNK), jnp.int32),
            pltpu.VMEM((CHUNK, DH), jnp.float32),
            pltpu.VMEM((CHUNK, DH), jnp.float32),
            pltpu.SemaphoreType.DMA,
            pltpu.SemaphoreType.DMA,
        ],
    )


_BLK = 1024


def _tc1_body(x_ref, w_ref, b_ref, cnt_ref, y_ref, dinv_ref):
    i = pl.program_id(0)
    xw = jnp.dot(x_ref[...], w_ref[...], preferred_element_type=jnp.float32)
    xw = xw + b_ref[...]
    ones = jnp.ones((32, 1), jnp.float32)
    deg = lax.dot_general(cnt_ref[...], ones, (((0,), (0,)), ((), ())),
                          preferred_element_type=jnp.float32)
    rowid = i * _BLK + lax.broadcasted_iota(jnp.int32, (_BLK, 1), 0)
    deg = deg + jnp.where(rowid < N, 1.0, 0.0)
    dinv = jnp.where(deg > 0, lax.rsqrt(deg), 0.0)
    y_ref[...] = xw * dinv
    dinv_ref[...] = dinv


def _tc2_body(s1_ref, dinv_ref, w_ref, b_ref, y_ref):
    dinv = dinv_ref[...]
    h = jnp.maximum(s1_ref[...] * dinv, 0.0)
    xw = jnp.dot(h, w_ref[...], preferred_element_type=jnp.float32)
    xw = xw + b_ref[...]
    y_ref[...] = xw * dinv


def _tc3_body(s2_ref, dinv_ref, k_ref, wk_ref, bk_ref, alpha_ref, o_ref):
    x2 = s2_ref[...] * dinv_ref[...]
    kp = jnp.dot(k_ref[...], wk_ref[...], preferred_element_type=jnp.float32)
    kp = kp + bk_ref[...]
    a = alpha_ref[0, 0]
    o_ref[...] = a * x2 + (1.0 - a) * kp


def kernel(graph_feature, k_node_pred, edge_index, W1, b1, W2, b2, Wk, bk, alpha):
    i32 = jnp.int32
    ei = edge_index.astype(i32)
    dst_a = ei[1].reshape(32, EA_PER_TILE)
    src_b = ei[0].reshape(16, NCHUNK, CHUNK)
    dst_b = ei[1].reshape(16, NCHUNK, CHUNK)

    x_pad = jnp.pad(graph_feature, ((0, N_P - N), (0, 0)))
    b1r = b1.reshape(1, D)
    b2r = b2.reshape(1, D)
    bkr = bk.reshape(1, D)
    alpha_arr = jnp.reshape(alpha, (1, 1)).astype(jnp.float32)

    cnt = _deg_call()(dst_a)

    grid = N_P // _BLK
    y1, dinv = pl.pallas_call(
        _tc1_body,
        grid=(grid,),
        in_specs=[
            pl.BlockSpec((_BLK, D), lambda i: (i, 0)),
            pl.BlockSpec((D, D), lambda i: (0, 0)),
            pl.BlockSpec((1, D), lambda i: (0, 0)),
            pl.BlockSpec((32, _BLK), lambda i: (0, i)),
        ],
        out_specs=[
            pl.BlockSpec((_BLK, D), lambda i: (i, 0)),
            pl.BlockSpec((_BLK, 1), lambda i: (i, 0)),
        ],
        out_shape=[
            jax.ShapeDtypeStruct((N_P, D), jnp.float32),
            jax.ShapeDtypeStruct((N_P, 1), jnp.float32),
        ],
    )(x_pad, W1, b1r, cnt)

    s1 = _segsum_call()(y1, src_b, dst_b)

    y2 = pl.pallas_call(
        _tc2_body,
        grid=(grid,),
        in_specs=[
            pl.BlockSpec((_BLK, D), lambda i: (i, 0)),
            pl.BlockSpec((_BLK, 1), lambda i: (i, 0)),
            pl.BlockSpec((D, D), lambda i: (0, 0)),
            pl.BlockSpec((1, D), lambda i: (0, 0)),
        ],
        out_specs=pl.BlockSpec((_BLK, D), lambda i: (i, 0)),
        out_shape=jax.ShapeDtypeStruct((N_P, D), jnp.float32),
    )(s1, dinv, W2, b2r)

    s2 = _segsum_call()(y2, src_b, dst_b)

    blk3 = 2000
    out = pl.pallas_call(
        _tc3_body,
        grid=(N // blk3,),
        in_specs=[
            pl.BlockSpec((blk3, D), lambda i: (i, 0)),
            pl.BlockSpec((blk3, 1), lambda i: (i, 0)),
            pl.BlockSpec((blk3, 64), lambda i: (i, 0)),
            pl.BlockSpec((64, D), lambda i: (0, 0)),
            pl.BlockSpec((1, D), lambda i: (0, 0)),
            pl.BlockSpec((1, 1), lambda i: (0, 0)),
        ],
        out_specs=pl.BlockSpec((blk3, D), lambda i: (i, 0)),
        out_shape=jax.ShapeDtypeStruct((N, D), jnp.float32),
    )(s2, dinv, k_node_pred, Wk, bkr, alpha_arr)

    return out
